# baseline (speedup 1.0000x reference)
import functools

import jax
import jax.numpy as jnp
from jax import lax
from jax.experimental import pallas as pl
from jax.experimental.pallas import tpu as pltpu
from jax.experimental.pallas import tpu_sc as plsc

NE = 1000000
NR = 1000
TCOLS = 4096
SPLIT_E = 124 * TCOLS
SPLIT_R = 512
DIM = 64
B = 1024
N = 200
MARGIN = 2.0
EPS = 1e-6

NC = 2
NS = 16
NW = NC * NS

BN = B * N
ROWS_PER_W = BN // NW
CH = 128
N_CH = ROWS_PER_W // CH
SMALL_PER_W = B // NW

_SC_PARAMS = pltpu.CompilerParams(use_tc_tiling_on_sc=True,
                                  needs_layout_passes=False)
_f32 = jnp.float32


def _tt_body(x1_ref, x2_ref, out_ref):
    out_ref[...] = jnp.concatenate(
        [x1_ref[...].T, x2_ref[...].T], axis=1)


def _transpose_pack(table_t, split, cols):
    nblk = split // cols
    n = table_t.shape[1]
    last = (n - 1) // cols

    def in2_map(i, _n=nblk, _last=last):
        return (0, jnp.minimum(i + _n, _last))

    return pl.pallas_call(
        _tt_body,
        grid=(nblk,),
        in_specs=[
            pl.BlockSpec((DIM, cols), lambda i: (0, i)),
            pl.BlockSpec((DIM, cols), in2_map),
        ],
        out_specs=pl.BlockSpec((cols, 2 * DIM), lambda i: (i, 0)),
        out_shape=jax.ShapeDtypeStruct((split, 2 * DIM), _f32),
    )(table_t, table_t)


def _sc_small_gather(u_idx, r_idx, embp, rdp, rb1p, rb2p):
    mesh = plsc.VectorSubcoreMesh(core_axis_name="c", subcore_axis_name="s")

    @functools.partial(
        pl.kernel,
        out_type=(
            jax.ShapeDtypeStruct((B, 2 * DIM), _f32),
            jax.ShapeDtypeStruct((B, 2 * DIM), _f32),
            jax.ShapeDtypeStruct((B, 2 * DIM), _f32),
            jax.ShapeDtypeStruct((B, 2 * DIM), _f32),
        ),
        mesh=mesh,
        compiler_params=_SC_PARAMS,
        scratch_types=[
            pltpu.VMEM((SMALL_PER_W,), jnp.int32),
            pltpu.VMEM((SMALL_PER_W,), jnp.int32),
            pltpu.VMEM((SMALL_PER_W, 2 * DIM), _f32),
            pltpu.SemaphoreType.DMA,
        ],
    )
    def k(u_idx_hbm, r_idx_hbm, emb_hbm, rd_hbm, rb1_hbm, rb2_hbm,
          head_out, rd_out, rb1_out, rb2_out, sidx_v, pidx_v, srows_v, sem):
        wid = lax.axis_index("s") * NC + lax.axis_index("c")
        sbase = pl.multiple_of(wid * SMALL_PER_W, 8)

        def mod_idx(m):
            for kk in range(SMALL_PER_W // 16):
                sl = pl.ds(kk * 16, 16)
                pidx_v[sl] = lax.rem(sidx_v[sl], m)

        pltpu.sync_copy(u_idx_hbm.at[pl.ds(sbase, SMALL_PER_W)], sidx_v)
        mod_idx(jnp.int32(SPLIT_E))
        pltpu.async_copy(emb_hbm.at[pidx_v], srows_v, sem).wait()
        pltpu.sync_copy(srows_v, head_out.at[pl.ds(sbase, SMALL_PER_W)])

        pltpu.sync_copy(r_idx_hbm.at[pl.ds(sbase, SMALL_PER_W)], sidx_v)
        mod_idx(jnp.int32(SPLIT_R))
        pltpu.async_copy(rd_hbm.at[pidx_v], srows_v, sem).wait()
        pltpu.sync_copy(srows_v, rd_out.at[pl.ds(sbase, SMALL_PER_W)])
        pltpu.async_copy(rb1_hbm.at[pidx_v], srows_v, sem).wait()
        pltpu.sync_copy(srows_v, rb1_out.at[pl.ds(sbase, SMALL_PER_W)])
        pltpu.async_copy(rb2_hbm.at[pidx_v], srows_v, sem).wait()
        pltpu.sync_copy(srows_v, rb2_out.at[pl.ds(sbase, SMALL_PER_W)])

    return k(u_idx, r_idx, embp, rdp, rb1p, rb2p)


HB = 128


def _tc_head_body(head_ref, rd_ref, rb1_ref, rb2_ref, uh_ref, rh_ref, out_ref):
    def pick(x2, hsel):
        return jnp.where(hsel, x2[:, DIM:], x2[:, :DIM])

    uh = uh_ref[...] != 0
    rh = rh_ref[...] != 0

    def expmap0(u):
        nrm = jnp.maximum(jnp.sqrt(jnp.sum(u * u, axis=-1, keepdims=True)), 1e-12)
        return jnp.tanh(nrm) * u / nrm

    def mobius_add(x, y):
        x2 = jnp.sum(x * x, axis=-1, keepdims=True)
        y2 = jnp.sum(y * y, axis=-1, keepdims=True)
        xy = jnp.sum(x * y, axis=-1, keepdims=True)
        num = (1.0 + 2.0 * xy + y2) * x + (1.0 - x2) * y
        den = 1.0 + 2.0 * xy + x2 * y2
        return num / jnp.maximum(den, 1e-12)

    h = expmap0(pick(head_ref[...], uh))
    b1 = expmap0(pick(rb1_ref[...], rh))
    b2 = expmap0(pick(rb2_ref[...], rh))
    h = mobius_add(h, b1)

    r = pick(rd_ref[...], rh)
    lane = lax.broadcasted_iota(jnp.int32, (HB, DIM), 1)
    even = (lane % 2) == 0

    def swap_pairs(x):
        return jnp.where(even, jnp.roll(x, -1, axis=-1), jnp.roll(x, 1, axis=-1))

    sr = swap_pairs(r)
    pn = jnp.maximum(jnp.sqrt(r * r + sr * sr), 1e-12)
    gg = r / pn
    sgg = swap_pairs(gg)
    sh = swap_pairs(h)
    h = jnp.where(even, gg, sgg) * h + jnp.where(even, -sgg, gg) * sh

    h = mobius_add(h, b2)
    out_ref[...] = h


def _tc_head(head_rows, rd_rows, rb1_rows, rb2_rows, u_half, r_half):
    return pl.pallas_call(
        _tc_head_body,
        grid=(B // HB,),
        in_specs=[pl.BlockSpec((HB, 2 * DIM), lambda i: (i, 0))] * 4
        + [pl.BlockSpec((HB, 1), lambda i: (i, 0))] * 2,
        out_specs=pl.BlockSpec((HB, DIM), lambda i: (i, 0)),
        out_shape=jax.ShapeDtypeStruct((B, DIM), _f32),
    )(head_rows, rd_rows, rb1_rows, rb2_rows, u_half, r_half)


def _sc_tail_dots(v_idx_flat, embp, head_flat):
    mesh = plsc.VectorSubcoreMesh(core_axis_name="c", subcore_axis_name="s")

    @functools.partial(
        pl.kernel,
        out_type=(
            jax.ShapeDtypeStruct((BN,), _f32),
            jax.ShapeDtypeStruct((BN,), _f32),
        ),
        mesh=mesh,
        compiler_params=_SC_PARAMS,
        scratch_types=[
            pltpu.VMEM((B * DIM,), _f32),
            pltpu.VMEM((ROWS_PER_W,), jnp.int32),
            pltpu.VMEM((CH,), jnp.int32),
            pltpu.VMEM((CH,), jnp.int32),
            pltpu.VMEM((CH, 2 * DIM), _f32),
            pltpu.VMEM((CH, 2 * DIM), _f32),
            pltpu.VMEM((ROWS_PER_W,), _f32),
            pltpu.VMEM((ROWS_PER_W,), _f32),
            pltpu.SemaphoreType.DMA,
            pltpu.SemaphoreType.DMA,
            pltpu.SemaphoreType.DMA,
        ],
    )
    def k(v_idx_hbm, emb_hbm, head_hbm, see_out, she_out,
          head_v, idx_w, idxb0, idxb1, rows0, rows1, see_v, she_v,
          semh, sem0, sem1):
        wid = lax.axis_index("s") * NC + lax.axis_index("c")
        w0 = pl.multiple_of(wid * ROWS_PER_W, 8)

        cph = pltpu.async_copy(head_hbm, head_v, semh)
        pltpu.sync_copy(v_idx_hbm.at[pl.ds(w0, ROWS_PER_W)], idx_w)
        cph.wait()

        iota = lax.iota(jnp.int32, 16)

        def fire(c, idxb, rows_v, sem):
            for kk in range(CH // 16):
                sl = pl.ds(kk * 16, 16)
                idxb[sl] = lax.rem(idx_w[pl.ds(c * CH + kk * 16, 16)],
                                   jnp.int32(SPLIT_E))
            return pltpu.async_copy(emb_hbm.at[idxb], rows_v, sem)

        def compute(c, rows_v):
            for g in range(8):
                row_g = w0 + c * CH + g * 16 + iota
                h_base = (row_g // N) * DIM
                e_row = g * 16 + iota
                v_raw = idx_w[pl.ds(c * CH + g * 16, 16)]
                half = jnp.where(v_raw >= SPLIT_E, DIM, 0).astype(jnp.int32)

                def dstep(i, carry):
                    a0, a1, b0, b1 = carry
                    s0 = i * 8
                    for u in range(8):
                        col = (iota + (s0 + u)) & (DIM - 1)
                        ev = plsc.load_gather(rows_v, [e_row, half + col])
                        hv = plsc.load_gather(head_v, [h_base + col])
                        if u % 2 == 0:
                            a0 = ev * ev + a0
                            b0 = hv * ev + b0
                        else:
                            a1 = ev * ev + a1
                            b1 = hv * ev + b1
                    return a0, a1, b0, b1

                z = jnp.zeros((16,), _f32)
                a0, a1, b0, b1 = lax.fori_loop(0, DIM // 8, dstep, (z, z, z, z))
                off = c * CH + g * 16
                see_v[pl.ds(off, 16)] = a0 + a1
                she_v[pl.ds(off, 16)] = b0 + b1

        def drain(idxb, rows_v, sem):
            pltpu.make_async_copy(emb_hbm.at[idxb], rows_v, sem).wait()

        fire(0, idxb0, rows0, sem0)

        def body(i, _):
            c0 = i * 2
            fire(c0 + 1, idxb1, rows1, sem1)
            drain(idxb0, rows0, sem0)
            compute(c0, rows0)

            @pl.when(i < N_CH // 2 - 1)
            def _():
                fire(c0 + 2, idxb0, rows0, sem0)

            drain(idxb1, rows1, sem1)
            compute(c0 + 1, rows1)
            return 0

        lax.fori_loop(0, N_CH // 2, body, 0)

        pltpu.sync_copy(see_v, see_out.at[pl.ds(w0, ROWS_PER_W)])
        pltpu.sync_copy(she_v, she_out.at[pl.ds(w0, ROWS_PER_W)])

    return k(v_idx_flat, embp, head_flat)


FB = 128


def _tc_final_body(see_ref, she_ref, head_ref, sig_ref, bh_ref, bt_ref, out_ref):
    h = head_ref[...]
    hh = jnp.sum(h * h, axis=-1, keepdims=True)
    hn = jnp.maximum(jnp.sqrt(hh), EPS)
    hd2 = hh / (hn * hn)

    s_ee = see_ref[...]
    s_he = she_ref[...]
    n = jnp.maximum(jnp.sqrt(s_ee), 1e-12)
    th = jnp.tanh(n)
    scale = th / n
    tt = scale * scale * s_ee

    num_t = hd2 - 2.0 * (scale / hn) * s_he + tt
    den_t = jnp.maximum(1.0 - tt, EPS)
    dist_t = jnp.log(jnp.maximum(num_t / den_t, EPS))

    inv = 1.0 / jnp.maximum(th, EPS)
    num_h = tt * inv * inv - 2.0 * (scale * inv) * s_he + hh
    den_h = jnp.maximum(1.0 - hh, EPS)
    dist_h = jnp.log(jnp.maximum(num_h / den_h, EPS))

    sig = jax.nn.sigmoid(sig_ref[...])
    dist = sig * dist_t + (1.0 - sig) * dist_h
    out_ref[...] = MARGIN - dist + bh_ref[...] + bt_ref[...]


def _tc_final(see, she, head, sig_b, bh_b, bt_bn):
    return pl.pallas_call(
        _tc_final_body,
        grid=(B // FB,),
        in_specs=[
            pl.BlockSpec((FB, N), lambda i: (i, 0)),
            pl.BlockSpec((FB, N), lambda i: (i, 0)),
            pl.BlockSpec((FB, DIM), lambda i: (i, 0)),
            pl.BlockSpec((FB, 1), lambda i: (i, 0)),
            pl.BlockSpec((FB, 1), lambda i: (i, 0)),
            pl.BlockSpec((FB, N), lambda i: (i, 0)),
        ],
        out_specs=pl.BlockSpec((FB, N), lambda i: (i, 0)),
        out_shape=jax.ShapeDtypeStruct((B, N), _f32),
    )(see, she, head, sig_b, bh_b, bt_bn)


def kernel(u_idx, r_idx, v_idx, emb_entity, rel_diag, relation_bias_1,
           relation_bias_2, bias_head, bias_tail, sigma):
    u_idx = u_idx.astype(jnp.int32)
    r_idx = r_idx.astype(jnp.int32)
    v_flat = v_idx.astype(jnp.int32).reshape(BN)

    sig_b = jnp.take(sigma, r_idx, axis=0).reshape(B, 1)
    bh_b = jnp.take(bias_head, u_idx, axis=0).reshape(B, 1)
    bt_bn = jnp.take(bias_tail, v_flat, axis=0).reshape(B, N)

    embp = _transpose_pack(emb_entity.T, SPLIT_E, TCOLS)
    rdp = _transpose_pack(rel_diag.T, SPLIT_R, SPLIT_R)
    rb1p = _transpose_pack(relation_bias_1.T, SPLIT_R, SPLIT_R)
    rb2p = _transpose_pack(relation_bias_2.T, SPLIT_R, SPLIT_R)

    u_idx, r_idx, bt_bn, bh_b, sig_b = lax.optimization_barrier(
        (u_idx, r_idx, bt_bn, bh_b, sig_b))

    head_rows, rd_rows, rb1_rows, rb2_rows = _sc_small_gather(
        u_idx, r_idx, embp, rdp, rb1p, rb2p)
    u_half = (u_idx >= SPLIT_E).astype(jnp.int32).reshape(B, 1)
    r_half = (r_idx >= SPLIT_R).astype(jnp.int32).reshape(B, 1)
    head = _tc_head(head_rows, rd_rows, rb1_rows, rb2_rows, u_half, r_half)
    see, she = _sc_tail_dots(v_flat, embp, head.reshape(B * DIM))

    return _tc_final(see.reshape(B, N), she.reshape(B, N), head, sig_b, bh_b,
                     bt_bn)


# speedup vs baseline: 1.0476x; 1.0476x over previous
import functools

import jax
import jax.numpy as jnp
from jax import lax
from jax.experimental import pallas as pl
from jax.experimental.pallas import tpu as pltpu
from jax.experimental.pallas import tpu_sc as plsc

NE = 1000000
NR = 1000
TCOLS = 4096
SPLIT_E = 124 * TCOLS
SPLIT_R = 512
DIM = 64
B = 1024
N = 200
MARGIN = 2.0
EPS = 1e-6

NC = 2
NS = 16
NW = NC * NS

BN = B * N
ROWS_PER_W = BN // NW
CH = 128
N_CH = ROWS_PER_W // CH
SMALL_PER_W = B // NW

_SC_PARAMS = pltpu.CompilerParams(use_tc_tiling_on_sc=True,
                                  needs_layout_passes=False)
_f32 = jnp.float32


def _tt_body(x1_ref, x2_ref, out_ref):
    out_ref[...] = jnp.concatenate(
        [x1_ref[...].T, x2_ref[...].T], axis=1)


def _transpose_pack(table_t, split, cols):
    nblk = split // cols
    n = table_t.shape[1]
    last = (n - 1) // cols

    def in2_map(i, _n=nblk, _last=last):
        return (0, jnp.minimum(i + _n, _last))

    return pl.pallas_call(
        _tt_body,
        grid=(nblk,),
        in_specs=[
            pl.BlockSpec((DIM, cols), lambda i: (0, i)),
            pl.BlockSpec((DIM, cols), in2_map),
        ],
        out_specs=pl.BlockSpec((cols, 2 * DIM), lambda i: (i, 0)),
        out_shape=jax.ShapeDtypeStruct((split, 2 * DIM), _f32),
    )(table_t, table_t)


def _sc_small_gather(u_idx, r_idx, embp, rdp, rb1p, rb2p):
    mesh = plsc.VectorSubcoreMesh(core_axis_name="c", subcore_axis_name="s")

    @functools.partial(
        pl.kernel,
        out_type=(
            jax.ShapeDtypeStruct((B, 2 * DIM), _f32),
            jax.ShapeDtypeStruct((B, 2 * DIM), _f32),
            jax.ShapeDtypeStruct((B, 2 * DIM), _f32),
            jax.ShapeDtypeStruct((B, 2 * DIM), _f32),
        ),
        mesh=mesh,
        compiler_params=_SC_PARAMS,
        scratch_types=[
            pltpu.VMEM((SMALL_PER_W,), jnp.int32),
            pltpu.VMEM((SMALL_PER_W,), jnp.int32),
            pltpu.VMEM((SMALL_PER_W, 2 * DIM), _f32),
            pltpu.SemaphoreType.DMA,
        ],
    )
    def k(u_idx_hbm, r_idx_hbm, emb_hbm, rd_hbm, rb1_hbm, rb2_hbm,
          head_out, rd_out, rb1_out, rb2_out, sidx_v, pidx_v, srows_v, sem):
        wid = lax.axis_index("s") * NC + lax.axis_index("c")
        sbase = pl.multiple_of(wid * SMALL_PER_W, 8)

        def mod_idx(m):
            for kk in range(SMALL_PER_W // 16):
                sl = pl.ds(kk * 16, 16)
                pidx_v[sl] = lax.rem(sidx_v[sl], m)

        pltpu.sync_copy(u_idx_hbm.at[pl.ds(sbase, SMALL_PER_W)], sidx_v)
        mod_idx(jnp.int32(SPLIT_E))
        pltpu.async_copy(emb_hbm.at[pidx_v], srows_v, sem).wait()
        pltpu.sync_copy(srows_v, head_out.at[pl.ds(sbase, SMALL_PER_W)])

        pltpu.sync_copy(r_idx_hbm.at[pl.ds(sbase, SMALL_PER_W)], sidx_v)
        mod_idx(jnp.int32(SPLIT_R))
        pltpu.async_copy(rd_hbm.at[pidx_v], srows_v, sem).wait()
        pltpu.sync_copy(srows_v, rd_out.at[pl.ds(sbase, SMALL_PER_W)])
        pltpu.async_copy(rb1_hbm.at[pidx_v], srows_v, sem).wait()
        pltpu.sync_copy(srows_v, rb1_out.at[pl.ds(sbase, SMALL_PER_W)])
        pltpu.async_copy(rb2_hbm.at[pidx_v], srows_v, sem).wait()
        pltpu.sync_copy(srows_v, rb2_out.at[pl.ds(sbase, SMALL_PER_W)])

    return k(u_idx, r_idx, embp, rdp, rb1p, rb2p)


HB = 128


def _tc_head_body(head_ref, rd_ref, rb1_ref, rb2_ref, uh_ref, rh_ref, out_ref):
    def pick(x2, hsel):
        return jnp.where(hsel, x2[:, DIM:], x2[:, :DIM])

    uh = uh_ref[...] != 0
    rh = rh_ref[...] != 0

    def expmap0(u):
        nrm = jnp.maximum(jnp.sqrt(jnp.sum(u * u, axis=-1, keepdims=True)), 1e-12)
        return jnp.tanh(nrm) * u / nrm

    def mobius_add(x, y):
        x2 = jnp.sum(x * x, axis=-1, keepdims=True)
        y2 = jnp.sum(y * y, axis=-1, keepdims=True)
        xy = jnp.sum(x * y, axis=-1, keepdims=True)
        num = (1.0 + 2.0 * xy + y2) * x + (1.0 - x2) * y
        den = 1.0 + 2.0 * xy + x2 * y2
        return num / jnp.maximum(den, 1e-12)

    h = expmap0(pick(head_ref[...], uh))
    b1 = expmap0(pick(rb1_ref[...], rh))
    b2 = expmap0(pick(rb2_ref[...], rh))
    h = mobius_add(h, b1)

    r = pick(rd_ref[...], rh)
    lane = lax.broadcasted_iota(jnp.int32, (HB, DIM), 1)
    even = (lane % 2) == 0

    def swap_pairs(x):
        return jnp.where(even, jnp.roll(x, -1, axis=-1), jnp.roll(x, 1, axis=-1))

    sr = swap_pairs(r)
    pn = jnp.maximum(jnp.sqrt(r * r + sr * sr), 1e-12)
    gg = r / pn
    sgg = swap_pairs(gg)
    sh = swap_pairs(h)
    h = jnp.where(even, gg, sgg) * h + jnp.where(even, -sgg, gg) * sh

    h = mobius_add(h, b2)
    out_ref[...] = h


def _tc_head(head_rows, rd_rows, rb1_rows, rb2_rows, u_half, r_half):
    return pl.pallas_call(
        _tc_head_body,
        grid=(B // HB,),
        in_specs=[pl.BlockSpec((HB, 2 * DIM), lambda i: (i, 0))] * 4
        + [pl.BlockSpec((HB, 1), lambda i: (i, 0))] * 2,
        out_specs=pl.BlockSpec((HB, DIM), lambda i: (i, 0)),
        out_shape=jax.ShapeDtypeStruct((B, DIM), _f32),
    )(head_rows, rd_rows, rb1_rows, rb2_rows, u_half, r_half)


def _sc_tail_dots(v_idx_flat, embp, head_flat):
    mesh = plsc.VectorSubcoreMesh(core_axis_name="c", subcore_axis_name="s")

    @functools.partial(
        pl.kernel,
        out_type=(
            jax.ShapeDtypeStruct((BN,), _f32),
            jax.ShapeDtypeStruct((BN,), _f32),
        ),
        mesh=mesh,
        compiler_params=_SC_PARAMS,
        scratch_types=[
            pltpu.VMEM((B * DIM,), _f32),
            pltpu.VMEM((ROWS_PER_W,), jnp.int32),
            pltpu.VMEM((CH,), jnp.int32),
            pltpu.VMEM((CH,), jnp.int32),
            pltpu.VMEM((CH, 2 * DIM), _f32),
            pltpu.VMEM((CH, 2 * DIM), _f32),
            pltpu.VMEM((ROWS_PER_W,), _f32),
            pltpu.VMEM((ROWS_PER_W,), _f32),
            pltpu.SemaphoreType.DMA,
            pltpu.SemaphoreType.DMA,
            pltpu.SemaphoreType.DMA,
        ],
    )
    def k(v_idx_hbm, emb_hbm, head_hbm, see_out, she_out,
          head_v, idx_w, idxb0, idxb1, rows0, rows1, see_v, she_v,
          semh, sem0, sem1):
        wid = lax.axis_index("s") * NC + lax.axis_index("c")
        w0 = pl.multiple_of(wid * ROWS_PER_W, 8)

        cph = pltpu.async_copy(head_hbm, head_v, semh)
        pltpu.sync_copy(v_idx_hbm.at[pl.ds(w0, ROWS_PER_W)], idx_w)
        cph.wait()

        iota = lax.iota(jnp.int32, 16)

        def fire(c, idxb, rows_v, sem):
            for kk in range(CH // 16):
                sl = pl.ds(kk * 16, 16)
                idxb[sl] = lax.rem(idx_w[pl.ds(c * CH + kk * 16, 16)],
                                   jnp.int32(SPLIT_E))
            return pltpu.async_copy(emb_hbm.at[idxb], rows_v, sem)

        def compute(c, rows_v):
            for g in range(8):
                row_g = w0 + c * CH + g * 16 + iota
                h_base = (row_g // N) * DIM
                e_row = g * 16 + iota
                v_raw = idx_w[pl.ds(c * CH + g * 16, 16)]
                half = jnp.where(v_raw >= SPLIT_E, DIM, 0).astype(jnp.int32)

                def dstep(i, carry):
                    a0, a1, b0, b1 = carry
                    s0 = i * 8
                    for u in range(8):
                        col = (iota + (s0 + u)) & (DIM - 1)
                        ev = plsc.load_gather(rows_v, [e_row, half + col])
                        hv = plsc.load_gather(head_v, [h_base + col])
                        if u % 2 == 0:
                            a0 = ev * ev + a0
                            b0 = hv * ev + b0
                        else:
                            a1 = ev * ev + a1
                            b1 = hv * ev + b1
                    return a0, a1, b0, b1

                z = jnp.zeros((16,), _f32)
                a0, a1, b0, b1 = lax.fori_loop(0, DIM // 8, dstep, (z, z, z, z))
                off = c * CH + g * 16
                see_v[pl.ds(off, 16)] = a0 + a1
                she_v[pl.ds(off, 16)] = b0 + b1

        def drain(idxb, rows_v, sem):
            pltpu.make_async_copy(emb_hbm.at[idxb], rows_v, sem).wait()

        fire(0, idxb0, rows0, sem0)

        def body(i, _):
            c0 = i * 2
            fire(c0 + 1, idxb1, rows1, sem1)
            drain(idxb0, rows0, sem0)
            compute(c0, rows0)

            @pl.when(i < N_CH // 2 - 1)
            def _():
                fire(c0 + 2, idxb0, rows0, sem0)

            drain(idxb1, rows1, sem1)
            compute(c0 + 1, rows1)
            return 0

        lax.fori_loop(0, N_CH // 2, body, 0)

        pltpu.sync_copy(see_v, see_out.at[pl.ds(w0, ROWS_PER_W)])
        pltpu.sync_copy(she_v, she_out.at[pl.ds(w0, ROWS_PER_W)])

    return k(v_idx_flat, embp, head_flat)


FB = 128


def _tc_final_body(see_ref, she_ref, head_ref, sig_ref, bh_ref, bt_ref, out_ref):
    h = head_ref[...]
    hh = jnp.sum(h * h, axis=-1, keepdims=True)
    hn = jnp.maximum(jnp.sqrt(hh), EPS)
    hd2 = hh / (hn * hn)

    s_ee = see_ref[...]
    s_he = she_ref[...]
    n = jnp.maximum(jnp.sqrt(s_ee), 1e-12)
    th = jnp.tanh(n)
    scale = th / n
    tt = scale * scale * s_ee

    num_t = hd2 - 2.0 * (scale / hn) * s_he + tt
    den_t = jnp.maximum(1.0 - tt, EPS)
    dist_t = jnp.log(jnp.maximum(num_t / den_t, EPS))

    inv = 1.0 / jnp.maximum(th, EPS)
    num_h = tt * inv * inv - 2.0 * (scale * inv) * s_he + hh
    den_h = jnp.maximum(1.0 - hh, EPS)
    dist_h = jnp.log(jnp.maximum(num_h / den_h, EPS))

    sig = jax.nn.sigmoid(sig_ref[...])
    dist = sig * dist_t + (1.0 - sig) * dist_h
    out_ref[...] = MARGIN - dist + bh_ref[...] + bt_ref[...]


def _tc_final(see, she, head, sig_b, bh_b, bt_bn):
    return pl.pallas_call(
        _tc_final_body,
        grid=(B // FB,),
        in_specs=[
            pl.BlockSpec((FB, N), lambda i: (i, 0)),
            pl.BlockSpec((FB, N), lambda i: (i, 0)),
            pl.BlockSpec((FB, DIM), lambda i: (i, 0)),
            pl.BlockSpec((FB, 1), lambda i: (i, 0)),
            pl.BlockSpec((FB, 1), lambda i: (i, 0)),
            pl.BlockSpec((FB, N), lambda i: (i, 0)),
        ],
        out_specs=pl.BlockSpec((FB, N), lambda i: (i, 0)),
        out_shape=jax.ShapeDtypeStruct((B, N), _f32),
    )(see, she, head, sig_b, bh_b, bt_bn)


def kernel(u_idx, r_idx, v_idx, emb_entity, rel_diag, relation_bias_1,
           relation_bias_2, bias_head, bias_tail, sigma):
    u_idx = u_idx.astype(jnp.int32)
    r_idx = r_idx.astype(jnp.int32)
    v_flat = v_idx.astype(jnp.int32).reshape(BN)

    sig_b = jnp.take(sigma, r_idx, axis=0).reshape(B, 1)
    bh_b = jnp.take(bias_head, u_idx, axis=0).reshape(B, 1)
    bt_bn = jnp.take(bias_tail, v_flat, axis=0).reshape(B, N)

    embp = _transpose_pack(emb_entity.T, SPLIT_E, TCOLS)
    rdp = _transpose_pack(rel_diag.T, SPLIT_R, SPLIT_R)
    rb1p = _transpose_pack(relation_bias_1.T, SPLIT_R, SPLIT_R)
    rb2p = _transpose_pack(relation_bias_2.T, SPLIT_R, SPLIT_R)

    head_rows, rd_rows, rb1_rows, rb2_rows = _sc_small_gather(
        u_idx, r_idx, embp, rdp, rb1p, rb2p)
    u_half = (u_idx >= SPLIT_E).astype(jnp.int32).reshape(B, 1)
    r_half = (r_idx >= SPLIT_R).astype(jnp.int32).reshape(B, 1)
    head = _tc_head(head_rows, rd_rows, rb1_rows, rb2_rows, u_half, r_half)
    see, she = _sc_tail_dots(v_flat, embp, head.reshape(B * DIM))

    return _tc_final(see.reshape(B, N), she.reshape(B, N), head, sig_b, bh_b,
                     bt_bn)
